# baseline (speedup 1.0000x reference)
import jax
import jax.numpy as jnp
from jax.experimental import pallas as pl
from jax.experimental.pallas import tpu as pltpu


_CH = 4


def _fill_and_copy_kernel(table_ref, out_ref, buf0, buf1, sem):
    N, S, HWD, E = out_ref.shape
    G = S // _CH
    bufs = (buf0, buf1)

    def copies(g):
        buf = bufs[g % 2]
        return [
            pltpu.make_async_copy(
                buf, out_ref.at[n, pl.ds(g * _CH, _CH)], sem.at[g % 2]
            )
            for n in range(N)
        ]

    for g in range(G):
        if g >= 2:
            for c in copies(g - 2):
                c.wait()
        rows = table_ref[pl.ds(g * _CH, _CH), :]
        bufs[g % 2][...] = jnp.broadcast_to(rows[:, None, :], (_CH, HWD, E))
        for c in copies(g):
            c.start()
    for g in range(max(G - 2, 0), G):
        for c in copies(g):
            c.wait()


def kernel(x, table):
    N, S, H, W, D = x.shape
    T, E = table.shape
    HWD = H * W * D

    out = pl.pallas_call(
        _fill_and_copy_kernel,
        in_specs=[pl.BlockSpec(memory_space=pltpu.VMEM)],
        out_specs=pl.BlockSpec(memory_space=pl.ANY),
        out_shape=jax.ShapeDtypeStruct((N, S, HWD, E), table.dtype),
        scratch_shapes=[
            pltpu.VMEM((_CH, HWD, E), table.dtype),
            pltpu.VMEM((_CH, HWD, E), table.dtype),
            pltpu.SemaphoreType.DMA((2,)),
        ],
    )(table)
    return out.reshape(N, S, H, W, D, E)


# speedup vs baseline: 1.0012x; 1.0012x over previous
import jax
import jax.numpy as jnp
from jax.experimental import pallas as pl
from jax.experimental.pallas import tpu as pltpu


_CH = 8


def _fill_and_copy_kernel(table_ref, out_ref, buf0, buf1, sem):
    N, S, HWD, E = out_ref.shape
    G = S // _CH
    bufs = (buf0, buf1)

    def copies(g):
        buf = bufs[g % 2]
        return [
            pltpu.make_async_copy(
                buf, out_ref.at[n, pl.ds(g * _CH, _CH)], sem.at[g % 2]
            )
            for n in range(N)
        ]

    for g in range(G):
        if g >= 2:
            for c in copies(g - 2):
                c.wait()
        rows = table_ref[pl.ds(g * _CH, _CH), :]
        bufs[g % 2][...] = jnp.broadcast_to(rows[:, None, :], (_CH, HWD, E))
        for c in copies(g):
            c.start()
    for g in range(max(G - 2, 0), G):
        for c in copies(g):
            c.wait()


def kernel(x, table):
    N, S, H, W, D = x.shape
    T, E = table.shape
    HWD = H * W * D

    out = pl.pallas_call(
        _fill_and_copy_kernel,
        in_specs=[pl.BlockSpec(memory_space=pltpu.VMEM)],
        out_specs=pl.BlockSpec(memory_space=pl.ANY),
        out_shape=jax.ShapeDtypeStruct((N, S, HWD, E), table.dtype),
        scratch_shapes=[
            pltpu.VMEM((_CH, HWD, E), table.dtype),
            pltpu.VMEM((_CH, HWD, E), table.dtype),
            pltpu.SemaphoreType.DMA((2,)),
        ],
    )(table)
    return out.reshape(N, S, H, W, D, E)
